# baseline (speedup 1.0000x reference)
import functools

import jax
import jax.numpy as jnp
from jax import lax
from jax.experimental import pallas as pl
from jax.experimental.pallas import tpu as pltpu
from jax.experimental.pallas import tpu_sc as plsc

B, P, N, D = 8, 64, 1024, 64
H = N // P
MAX_LEN = 1000
YROWS = B * P * D * (P // 8)
DC = 32


def _prep_body(idx_ref, table_ref, o_ref):
    rows = [table_ref[pl.ds(idx_ref[q], 1), :] for q in range(P)]
    t64 = jnp.concatenate(rows, axis=0)
    tt = t64.T
    o_ref[...] = jnp.concatenate([tt, tt], axis=1)


def _tc_prep(idx, table):
    return pl.pallas_call(
        _prep_body,
        in_specs=[
            pl.BlockSpec(memory_space=pltpu.SMEM),
            pl.BlockSpec((MAX_LEN, D), lambda: (0, 0)),
        ],
        out_specs=pl.BlockSpec((D, 128), lambda: (0, 0)),
        out_shape=jax.ShapeDtypeStruct((D, 128), jnp.float32),
    )(idx, table)


def _sc_body(x_hbm, wpe_hbm, y_hbm, wpe_v, xb0, xb1, st0, st1, pattern,
             si0, si1, sem_pe, sem_in0, sem_in1, sem_out0, sem_out1):
    wid = lax.axis_index("s") * 2 + lax.axis_index("c")
    iota = lax.iota(jnp.int32, 16)
    iota2 = iota * 2

    xbufs = (xb0, xb1)
    stages = (st0, st1)
    sidxs = (si0, si1)
    sems_in = (sem_in0, sem_in1)
    sems_out = (sem_out0, sem_out1)

    pltpu.async_copy(wpe_hbm, wpe_v, sem_pe).wait()

    for kb in range(8):
        j = iota + kb * 16
        pattern[0, pl.ds(kb * 16, 16)] = ((j >> 1) << 9) + ((j & 1) << 3)

    def issue_x(b, pg, dc, par):
        return pltpu.async_copy(
            x_hbm.at[b, pl.ds(pg * 8, 8), pl.ds(dc * 2, 2), :],
            xbufs[par], sems_in[par])

    def wait_x(par):
        pltpu.make_async_copy(
            x_hbm.at[0, pl.ds(0, 8), pl.ds(0, 2), :],
            xbufs[par], sems_in[par]).wait()

    def build_sidx(b, dc, pg, par):
        basev = b * 32768 + dc * 16 + pg
        for kb in range(8):
            sidxs[par][0, pl.ds(kb * 16, 16)] = (
                pattern[0, pl.ds(kb * 16, 16)] + basev)

    def issue_scatter(par):
        return pltpu.async_copy(stages[par],
                                y_hbm.at[sidxs[par].at[0]], sems_out[par])

    def wait_scatter(par):
        pltpu.make_async_copy(stages[par],
                              y_hbm.at[sidxs[par].at[0]],
                              sems_out[par]).wait()

    def shuffle(dc, par):
        @plsc.parallel_loop(0, 64, unroll=8)
        def shuf(i):
            pl_ = i >> 3
            dl = (i >> 2) & 1
            qb = i & 3
            wv = wpe_v[dc * 2 + dl, pl.ds(qb * 16, 16)]
            rowv = iota2 + (qb * 32 + dl)
            cbase = pl_ * 16
            for h in range(H):
                xv = xbufs[par][pl_, dl, pl.ds(h * 64 + qb * 16, 16)]
                colv = jnp.full((16,), cbase + h, jnp.int32)
                plsc.store_scatter(stages[par], [rowv, colv], xv + wv)

    def unit(uu):
        u = wid * 2 + uu
        b = u // 8
        pg = u % 8
        issue_x(b, pg, 0, 0)
        build_sidx(b, 0, pg, 0)
        issue_scatter(0)
        build_sidx(b, 1, pg, 1)
        issue_scatter(1)

        def body(dcp, c):
            for par in range(2):
                dc = dcp * 2 + par
                if par == 0:
                    issue_x(b, pg, dc + 1, 1)
                else:
                    nxt = jnp.minimum(dc + 1, DC - 1)
                    issue_x(b, pg, nxt, 0)
                wait_x(par)
                wait_scatter(par)
                shuffle(dc, par)
                build_sidx(b, dc, pg, par)
                issue_scatter(par)
            return c

        lax.fori_loop(0, DC // 2, body, 0)
        wait_scatter(0)
        wait_scatter(1)
        wait_x(0)

    unit(0)
    unit(1)


def _sc_main(x_t, wpe):
    mesh = plsc.VectorSubcoreMesh(core_axis_name="c", subcore_axis_name="s")
    kern = functools.partial(
        pl.kernel,
        mesh=mesh,
        out_type=jax.ShapeDtypeStruct((YROWS, 128), jnp.float32),
        scratch_types=[
            pltpu.VMEM((D, 128), jnp.float32),
            pltpu.VMEM((8, 2, N), jnp.float32),
            pltpu.VMEM((8, 2, N), jnp.float32),
            pltpu.VMEM((128, 128), jnp.float32),
            pltpu.VMEM((128, 128), jnp.float32),
            pltpu.VMEM((1, 128), jnp.int32),
            pltpu.VMEM((1, 128), jnp.int32),
            pltpu.VMEM((1, 128), jnp.int32),
            pltpu.SemaphoreType.DMA,
            pltpu.SemaphoreType.DMA,
            pltpu.SemaphoreType.DMA,
            pltpu.SemaphoreType.DMA,
            pltpu.SemaphoreType.DMA,
        ],
        compiler_params=pltpu.CompilerParams(needs_layout_passes=False),
    )(_sc_body)
    return kern(x_t, wpe)


@jax.jit
def kernel(input_data, index, position_embedding):
    idx = index.astype(jnp.int32)
    wpe = _tc_prep(idx, position_embedding)
    x_t = jnp.transpose(input_data, (0, 1, 3, 2))
    yv = _sc_main(x_t, wpe)
    out_t = yv.reshape(B, P, D, N)
    return jnp.transpose(out_t, (0, 3, 1, 2))


# speedup vs baseline: 1.1743x; 1.1743x over previous
import functools

import jax
import jax.numpy as jnp
from jax import lax
from jax.experimental import pallas as pl
from jax.experimental.pallas import tpu as pltpu
from jax.experimental.pallas import tpu_sc as plsc

B, P, N, D = 8, 64, 1024, 64
H = N // P
MAX_LEN = 1000
YROWS = B * P * D * (P // 8)
DC = 32


def _prep_body(idx_ref, table_ref, o_ref):
    rows = [table_ref[pl.ds(idx_ref[q], 1), :] for q in range(P)]
    t64 = jnp.concatenate(rows, axis=0)
    tt = t64.T
    o_ref[...] = jnp.concatenate([tt, tt], axis=1)


def _tc_prep(idx, table):
    return pl.pallas_call(
        _prep_body,
        in_specs=[
            pl.BlockSpec(memory_space=pltpu.SMEM),
            pl.BlockSpec((MAX_LEN, D), lambda: (0, 0)),
        ],
        out_specs=pl.BlockSpec((D, 128), lambda: (0, 0)),
        out_shape=jax.ShapeDtypeStruct((D, 128), jnp.float32),
    )(idx, table)


def _sc_body(x_hbm, wpe_hbm, y_hbm, wpe_v, xb0, xb1, st0, st1, pattern,
             si0, si1, sem_pe, sem_in0, sem_in1, sem_out0, sem_out1):
    wid = lax.axis_index("s") * 2 + lax.axis_index("c")
    iota = lax.iota(jnp.int32, 16)
    iota2 = iota * 2

    xbufs = (xb0, xb1)
    stages = (st0, st1)
    sidxs = (si0, si1)
    sems_in = (sem_in0, sem_in1)
    sems_out = (sem_out0, sem_out1)

    pltpu.async_copy(wpe_hbm, wpe_v, sem_pe).wait()

    for kb in range(8):
        j = iota + kb * 16
        pattern[0, pl.ds(kb * 16, 16)] = ((j >> 1) << 9) + ((j & 1) << 3)

    def issue_x(b, pg, dc, par):
        return pltpu.async_copy(
            x_hbm.at[b, pl.ds(pg * 8, 8), pl.ds(dc * 2, 2), :],
            xbufs[par], sems_in[par])

    def wait_x(par):
        pltpu.make_async_copy(
            x_hbm.at[0, pl.ds(0, 8), pl.ds(0, 2), :],
            xbufs[par], sems_in[par]).wait()

    def build_sidx(b, dc, pg, par):
        basev = b * 32768 + dc * 16 + pg
        for kb in range(8):
            sidxs[par][0, pl.ds(kb * 16, 16)] = (
                pattern[0, pl.ds(kb * 16, 16)] + basev)

    def issue_scatter(par):
        return pltpu.async_copy(stages[par],
                                y_hbm.at[sidxs[par].at[0]], sems_out[par])

    def wait_scatter(par):
        pltpu.make_async_copy(stages[par],
                              y_hbm.at[sidxs[par].at[0]],
                              sems_out[par]).wait()

    iota64 = iota * 64
    pl_splats = [jnp.full((16,), k, jnp.int32) for k in range(8)]

    def shuffle(dc, par):
        @plsc.parallel_loop(0, 128, unroll=4)
        def shuf(j):
            dl = j & 1
            q = j >> 1
            d = dc * 2 + dl
            wv = plsc.load_gather(
                wpe_v, [jnp.full((16,), d, jnp.int32),
                        jnp.full((16,), q, jnp.int32)])
            dlv = jnp.full((16,), dl, jnp.int32)
            cv = iota64 + q
            for pl_ in range(8):
                xv = plsc.load_gather(xbufs[par], [pl_splats[pl_], dlv, cv])
                stages[par][j, pl.ds(pl_ * 16, 16)] = xv + wv

    def unit(uu):
        u = wid * 2 + uu
        b = u // 8
        pg = u % 8
        issue_x(b, pg, 0, 0)
        build_sidx(b, 0, pg, 0)
        issue_scatter(0)
        build_sidx(b, 1, pg, 1)
        issue_scatter(1)

        def body(dcp, c):
            for par in range(2):
                dc = dcp * 2 + par
                if par == 0:
                    issue_x(b, pg, dc + 1, 1)
                else:
                    nxt = jnp.minimum(dc + 1, DC - 1)
                    issue_x(b, pg, nxt, 0)
                wait_x(par)
                wait_scatter(par)
                shuffle(dc, par)
                build_sidx(b, dc, pg, par)
                issue_scatter(par)
            return c

        lax.fori_loop(0, DC // 2, body, 0)
        wait_scatter(0)
        wait_scatter(1)
        wait_x(0)

    unit(0)
    unit(1)


def _sc_main(x_t, wpe):
    mesh = plsc.VectorSubcoreMesh(core_axis_name="c", subcore_axis_name="s")
    kern = functools.partial(
        pl.kernel,
        mesh=mesh,
        out_type=jax.ShapeDtypeStruct((YROWS, 128), jnp.float32),
        scratch_types=[
            pltpu.VMEM((D, 128), jnp.float32),
            pltpu.VMEM((8, 2, N), jnp.float32),
            pltpu.VMEM((8, 2, N), jnp.float32),
            pltpu.VMEM((128, 128), jnp.float32),
            pltpu.VMEM((128, 128), jnp.float32),
            pltpu.VMEM((1, 128), jnp.int32),
            pltpu.VMEM((1, 128), jnp.int32),
            pltpu.VMEM((1, 128), jnp.int32),
            pltpu.SemaphoreType.DMA,
            pltpu.SemaphoreType.DMA,
            pltpu.SemaphoreType.DMA,
            pltpu.SemaphoreType.DMA,
            pltpu.SemaphoreType.DMA,
        ],
        compiler_params=pltpu.CompilerParams(needs_layout_passes=False),
    )(_sc_body)
    return kern(x_t, wpe)


@jax.jit
def kernel(input_data, index, position_embedding):
    idx = index.astype(jnp.int32)
    wpe = _tc_prep(idx, position_embedding)
    x_t = jnp.transpose(input_data, (0, 1, 3, 2))
    yv = _sc_main(x_t, wpe)
    out_t = yv.reshape(B, P, D, N)
    return jnp.transpose(out_t, (0, 3, 1, 2))


# speedup vs baseline: 1.1802x; 1.0050x over previous
import functools

import jax
import jax.numpy as jnp
from jax import lax
from jax.experimental import pallas as pl
from jax.experimental.pallas import tpu as pltpu
from jax.experimental.pallas import tpu_sc as plsc

B, P, N, D = 8, 64, 1024, 64
H = N // P
MAX_LEN = 1000
YROWS = B * P * D * (P // 8)
DC = 32


def _prep_body(idx_ref, table_ref, o_ref):
    rows = [table_ref[pl.ds(idx_ref[q], 1), :] for q in range(P)]
    t64 = jnp.concatenate(rows, axis=0)
    tt = t64.T
    o_ref[...] = jnp.concatenate([tt, tt], axis=1)


def _tc_prep(idx, table):
    return pl.pallas_call(
        _prep_body,
        in_specs=[
            pl.BlockSpec(memory_space=pltpu.SMEM),
            pl.BlockSpec((MAX_LEN, D), lambda: (0, 0)),
        ],
        out_specs=pl.BlockSpec((D, 128), lambda: (0, 0)),
        out_shape=jax.ShapeDtypeStruct((D, 128), jnp.float32),
    )(idx, table)


def _sc_body(x_hbm, wpe_hbm, y_hbm, wpe_v, xb0, xb1, st0, st1, pattern,
             si0, si1, sem_pe, sem_in0, sem_in1, sem_out0, sem_out1):
    wid = lax.axis_index("s") * 2 + lax.axis_index("c")
    iota = lax.iota(jnp.int32, 16)

    xbufs = (xb0, xb1)
    stages = (st0, st1)
    sidxs = (si0, si1)
    sems_in = (sem_in0, sem_in1)
    sems_out = (sem_out0, sem_out1)

    pltpu.async_copy(wpe_hbm, wpe_v, sem_pe).wait()

    for kb in range(8):
        j = iota + kb * 16
        pattern[0, pl.ds(kb * 16, 16)] = ((j >> 1) << 9) + ((j & 1) << 3)

    def issue_x(b, pg, dc, par):
        return pltpu.async_copy(
            x_hbm.at[b, pl.ds(pg * 8, 8), pl.ds(dc * 2, 2), :],
            xbufs[par], sems_in[par])

    def wait_x(par):
        pltpu.make_async_copy(
            x_hbm.at[0, pl.ds(0, 8), pl.ds(0, 2), :],
            xbufs[par], sems_in[par]).wait()

    def build_sidx(b, dc, pg, par):
        basev = b * 32768 + dc * 16 + pg
        for kb in range(8):
            sidxs[par][0, pl.ds(kb * 16, 16)] = (
                pattern[0, pl.ds(kb * 16, 16)] + basev)

    def issue_scatter(par):
        return pltpu.async_copy(stages[par],
                                y_hbm.at[sidxs[par].at[0]], sems_out[par])

    def wait_scatter(par):
        pltpu.make_async_copy(stages[par],
                              y_hbm.at[sidxs[par].at[0]],
                              sems_out[par]).wait()

    iota64 = iota * 64
    pl_splats = [jnp.full((16,), k, jnp.int32) for k in range(8)]

    def shuffle(dc, par):
        @plsc.parallel_loop(0, 128, unroll=4)
        def shuf(j):
            dl = j & 1
            q = j >> 1
            d = dc * 2 + dl
            wv = plsc.load_gather(
                wpe_v, [jnp.full((16,), d, jnp.int32),
                        jnp.full((16,), q, jnp.int32)])
            dlv = jnp.full((16,), dl, jnp.int32)
            cv = iota64 + q
            for pl_ in range(8):
                xv = plsc.load_gather(xbufs[par], [pl_splats[pl_], dlv, cv])
                stages[par][j, pl.ds(pl_ * 16, 16)] = xv + wv

    def unit(uu):
        u = wid * 2 + uu
        b = u // 8
        pg = u % 8
        issue_x(b, pg, 0, 0)
        build_sidx(b, 0, pg, 0)
        issue_scatter(0)
        build_sidx(b, 1, pg, 1)
        issue_scatter(1)

        def body(dcp, c):
            for par in range(2):
                dc = dcp * 2 + par
                if par == 0:
                    issue_x(b, pg, dc + 1, 1)
                else:
                    nxt = jnp.minimum(dc + 1, DC - 1)
                    issue_x(b, pg, nxt, 0)
                wait_x(par)
                wait_scatter(par)
                shuffle(dc, par)
                build_sidx(b, dc, pg, par)
                issue_scatter(par)
            return c

        lax.fori_loop(0, DC // 2, body, 0)
        wait_scatter(0)
        wait_scatter(1)
        wait_x(0)

    unit(0)
    unit(1)


def _sc_main(x_t, wpe):
    mesh = plsc.VectorSubcoreMesh(core_axis_name="c", subcore_axis_name="s")
    kern = functools.partial(
        pl.kernel,
        mesh=mesh,
        out_type=jax.ShapeDtypeStruct((YROWS, 128), jnp.float32),
        scratch_types=[
            pltpu.VMEM((D, 128), jnp.float32),
            pltpu.VMEM((8, 2, N), jnp.float32),
            pltpu.VMEM((8, 2, N), jnp.float32),
            pltpu.VMEM((128, 128), jnp.float32),
            pltpu.VMEM((128, 128), jnp.float32),
            pltpu.VMEM((1, 128), jnp.int32),
            pltpu.VMEM((1, 128), jnp.int32),
            pltpu.VMEM((1, 128), jnp.int32),
            pltpu.SemaphoreType.DMA,
            pltpu.SemaphoreType.DMA,
            pltpu.SemaphoreType.DMA,
            pltpu.SemaphoreType.DMA,
            pltpu.SemaphoreType.DMA,
        ],
        compiler_params=pltpu.CompilerParams(needs_layout_passes=False),
    )(_sc_body)
    return kern(x_t, wpe)


@jax.jit
def kernel(input_data, index, position_embedding):
    idx = index.astype(jnp.int32)
    wpe = _tc_prep(idx, position_embedding)
    x_t = jnp.transpose(input_data, (0, 1, 3, 2))
    yv = _sc_main(x_t, wpe)
    out_t = yv.reshape(B, P, D, N)
    return jnp.transpose(out_t, (0, 3, 1, 2))
